# baseline (device time: 76963 ns/iter reference)
import jax
import jax.numpy as jnp
from jax import lax
from jax.experimental import pallas as pl
from jax.experimental.pallas import tpu as pltpu

N_DEV = 4
SQ = 2048
D_MODEL = 1024
HQ = 8
DH = 128
D_ATTN = HQ * DH
SCALE = 0.08838834764831843
N_GROUPS = 4
BLK = 64
M_BLOCKS = SQ // BLK // N_GROUPS
GROUP = M_BLOCKS * BLK
CHUNK = SQ // N_DEV
HALF = CHUNK // 2


def kernel(x, Wq, K_ext, V_ext, Wo):
    K2 = K_ext.reshape(M_BLOCKS, N_GROUPS, BLK, HQ, DH)
    V2 = V_ext.reshape(M_BLOCKS, N_GROUPS, BLK, HQ, DH)

    def body(x_ref, wq_ref, k_ref, v_ref, wo_ref, out_ref,
             wq_sl, wo_sl, kg, vg, comm_ref, stage_ref,
             copy_sems, send_sems, recv_sems):
        my = lax.axis_index("i")
        left = lax.rem(my + N_DEV - 1, N_DEV)
        right = lax.rem(my + 1, N_DEV)

        copies = [
            pltpu.make_async_copy(
                wq_ref.at[:, pl.ds(my * D_ATTN, D_ATTN)], wq_sl,
                copy_sems.at[0]),
            pltpu.make_async_copy(
                wo_ref.at[pl.ds(my * D_ATTN, D_ATTN), :], wo_sl,
                copy_sems.at[1]),
        ]
        for r in range(N_GROUPS):
            for h in range(HQ):
                copies.append(pltpu.make_async_copy(
                    k_ref.at[:, r, :, h, :], kg.at[r, h], copy_sems.at[2]))
                copies.append(pltpu.make_async_copy(
                    v_ref.at[:, r, :, h, :], vg.at[r, h], copy_sems.at[2]))
        for cp in copies:
            cp.start()

        barrier_sem = pltpu.get_barrier_semaphore()
        for nbr in (left, right):
            pl.semaphore_signal(
                barrier_sem, inc=1,
                device_id=(nbr,), device_id_type=pl.DeviceIdType.MESH,
            )
        pl.semaphore_wait(barrier_sem, 2)
        for cp in copies:
            cp.wait()

        def compute_half(c, off):
            row0 = c * CHUNK + off
            xh = x_ref[0, pl.ds(row0, HALF), :]
            q = jnp.dot(xh, wq_sl[...],
                        preferred_element_type=jnp.float32)
            q4 = q.reshape(N_GROUPS, BLK, HQ, DH)
            rows = []
            for r in range(N_GROUPS):
                qr = q4[r].transpose(1, 0, 2)
                kr = kg[r].reshape(HQ, GROUP, DH)
                vr = vg[r].reshape(HQ, GROUP, DH)
                s = lax.dot_general(
                    qr, kr, (((2,), (2,)), ((0,), (0,))),
                    preferred_element_type=jnp.float32) * SCALE
                s = s - jnp.max(s, axis=-1, keepdims=True)
                w = jnp.exp(s)
                w = w / jnp.sum(w, axis=-1, keepdims=True)
                ctx = lax.dot_general(
                    w, vr, (((2,), (1,)), ((0,), (0,))),
                    preferred_element_type=jnp.float32)
                rows.append(ctx.transpose(1, 0, 2).reshape(BLK, D_ATTN))
            ctx_half = jnp.concatenate(rows)
            out_ref[0, pl.ds(row0, HALF), :] = jnp.dot(
                ctx_half, wo_sl[...], preferred_element_type=jnp.float32)

        def cw_chunk(k):
            return lax.rem(my + 4 * N_DEV - k, N_DEV)

        def ccw_chunk(k):
            return lax.rem(my + k, N_DEV)

        def rs_rdma(s):
            cw = pltpu.make_async_remote_copy(
                src_ref=stage_ref.at[0],
                dst_ref=comm_ref.at[s],
                send_sem=send_sems.at[s],
                recv_sem=recv_sems.at[s],
                device_id=(right,),
                device_id_type=pl.DeviceIdType.MESH,
            )
            ccw = pltpu.make_async_remote_copy(
                src_ref=stage_ref.at[1],
                dst_ref=comm_ref.at[3 + s],
                send_sem=send_sems.at[3 + s],
                recv_sem=recv_sems.at[3 + s],
                device_id=(left,),
                device_id_type=pl.DeviceIdType.MESH,
            )
            return cw, ccw

        cw_d, ccw_d = rs_rdma(0)
        compute_half(cw_chunk(0), 0)
        stage_ref[0] = out_ref[0, pl.ds(cw_chunk(0) * CHUNK, HALF), :].astype(
            jnp.bfloat16)
        cw_d.start()
        compute_half(ccw_chunk(0), HALF)
        stage_ref[1] = out_ref[
            0, pl.ds(ccw_chunk(0) * CHUNK + HALF, HALF), :].astype(jnp.bfloat16)
        ccw_d.start()

        for s in range(N_DEV - 1):
            compute_half(cw_chunk(s + 1), 0)
            compute_half(ccw_chunk(s + 1), HALF)
            cw_d.wait()
            ccw_d.wait()
            tmp_cw = (out_ref[0, pl.ds(cw_chunk(s + 1) * CHUNK, HALF), :]
                      + comm_ref[s].astype(jnp.float32))
            tmp_ccw = (out_ref[0, pl.ds(ccw_chunk(s + 1) * CHUNK + HALF, HALF), :]
                       + comm_ref[3 + s].astype(jnp.float32))
            if s < N_DEV - 2:
                stage_ref[0] = tmp_cw.astype(jnp.bfloat16)
                stage_ref[1] = tmp_ccw.astype(jnp.bfloat16)
                cw_d, ccw_d = rs_rdma(s + 1)
                cw_d.start()
                ccw_d.start()
            else:
                stage_ref[2] = tmp_cw.astype(jnp.bfloat16)
                stage_ref[3] = tmp_ccw.astype(jnp.bfloat16)
            out_ref[0, pl.ds(cw_chunk(s + 1) * CHUNK, HALF), :] = tmp_cw
            out_ref[0, pl.ds(ccw_chunk(s + 1) * CHUNK + HALF, HALF), :] = tmp_ccw

        for t in range(N_DEV - 1):
            cw = pltpu.make_async_remote_copy(
                src_ref=stage_ref.at[2] if t == 0 else comm_ref.at[t - 1],
                dst_ref=comm_ref.at[t],
                send_sem=send_sems.at[6 + t],
                recv_sem=recv_sems.at[6 + t],
                device_id=(right,),
                device_id_type=pl.DeviceIdType.MESH,
            )
            ccw = pltpu.make_async_remote_copy(
                src_ref=stage_ref.at[3] if t == 0 else comm_ref.at[3 + t - 1],
                dst_ref=comm_ref.at[3 + t],
                send_sem=send_sems.at[9 + t],
                recv_sem=recv_sems.at[9 + t],
                device_id=(left,),
                device_id_type=pl.DeviceIdType.MESH,
            )
            cw.start()
            ccw.start()
            if t > 0:
                gc = lax.rem(my + 4 * N_DEV - (t - 1), N_DEV)
                bc = lax.rem(my + t - 1, N_DEV)
                out_ref[0, pl.ds(gc * CHUNK, HALF), :] = (
                    comm_ref[t - 1].astype(jnp.float32))
                out_ref[0, pl.ds(bc * CHUNK + HALF, HALF), :] = (
                    comm_ref[3 + t - 1].astype(jnp.float32))
            cw.wait()
            ccw.wait()
        t_last = N_DEV - 2
        gc = lax.rem(my + 4 * N_DEV - t_last, N_DEV)
        bc = lax.rem(my + t_last, N_DEV)
        out_ref[0, pl.ds(gc * CHUNK, HALF), :] = (
            comm_ref[t_last].astype(jnp.float32))
        out_ref[0, pl.ds(bc * CHUNK + HALF, HALF), :] = (
            comm_ref[3 + t_last].astype(jnp.float32))

    out_shape = jax.ShapeDtypeStruct((1, SQ, D_MODEL), jnp.float32)
    return pl.pallas_call(
        body,
        out_shape=out_shape,
        in_specs=[
            pl.BlockSpec(memory_space=pltpu.VMEM),
            pl.BlockSpec(memory_space=pltpu.MemorySpace.HBM),
            pl.BlockSpec(memory_space=pltpu.MemorySpace.HBM),
            pl.BlockSpec(memory_space=pltpu.MemorySpace.HBM),
            pl.BlockSpec(memory_space=pltpu.MemorySpace.HBM),
        ],
        out_specs=pl.BlockSpec(memory_space=pltpu.VMEM),
        scratch_shapes=[
            pltpu.VMEM((D_MODEL, D_ATTN), jnp.float32),
            pltpu.VMEM((D_ATTN, D_MODEL), jnp.float32),
            pltpu.VMEM((N_GROUPS, HQ, M_BLOCKS, BLK, DH), jnp.float32),
            pltpu.VMEM((N_GROUPS, HQ, M_BLOCKS, BLK, DH), jnp.float32),
            pltpu.VMEM((6, HALF, D_MODEL), jnp.bfloat16),
            pltpu.VMEM((4, HALF, D_MODEL), jnp.bfloat16),
            pltpu.SemaphoreType.DMA((3,)),
            pltpu.SemaphoreType.DMA((12,)),
            pltpu.SemaphoreType.DMA((12,)),
        ],
        compiler_params=pltpu.CompilerParams(collective_id=0),
    )(x, Wq, K2, V2, Wo)


# device time: 75007 ns/iter; 1.0261x vs baseline; 1.0261x over previous
import jax
import jax.numpy as jnp
from jax import lax
from jax.experimental import pallas as pl
from jax.experimental.pallas import tpu as pltpu

N_DEV = 4
SQ = 2048
D_MODEL = 1024
HQ = 8
DH = 128
D_ATTN = HQ * DH
SCALE = 0.08838834764831843
N_GROUPS = 4
BLK = 64
M_BLOCKS = SQ // BLK // N_GROUPS
GROUP = M_BLOCKS * BLK
CHUNK = SQ // N_DEV
HALF = CHUNK // 2


def kernel(x, Wq, K_ext, V_ext, Wo):
    K2 = K_ext.reshape(M_BLOCKS, N_GROUPS, BLK, HQ, DH)
    V2 = V_ext.reshape(M_BLOCKS, N_GROUPS, BLK, HQ, DH)

    def body(x_ref, wq_ref, k_ref, v_ref, wo_ref, out_ref,
             wq_sl, wo_sl, kg, vg, comm_ref, stage_ref,
             copy_sems, send_sems, recv_sems):
        my = lax.axis_index("i")
        left = lax.rem(my + N_DEV - 1, N_DEV)
        right = lax.rem(my + 1, N_DEV)

        wq_cp = pltpu.make_async_copy(
            wq_ref.at[:, pl.ds(my * D_ATTN, D_ATTN)], wq_sl, copy_sems.at[0])
        wo_cp = pltpu.make_async_copy(
            wo_ref.at[pl.ds(my * D_ATTN, D_ATTN), :], wo_sl, copy_sems.at[1])
        kv_copies = [[] for _ in range(N_GROUPS)]
        for r in range(N_GROUPS):
            for h in range(HQ):
                kv_copies[r].append(pltpu.make_async_copy(
                    k_ref.at[:, r, :, h, :], kg.at[r, h], copy_sems.at[2 + r]))
                kv_copies[r].append(pltpu.make_async_copy(
                    v_ref.at[:, r, :, h, :], vg.at[r, h], copy_sems.at[2 + r]))
        wq_cp.start()
        wo_cp.start()
        for r in range(N_GROUPS):
            for cp in kv_copies[r]:
                cp.start()

        barrier_sem = pltpu.get_barrier_semaphore()
        for nbr in (left, right):
            pl.semaphore_signal(
                barrier_sem, inc=1,
                device_id=(nbr,), device_id_type=pl.DeviceIdType.MESH,
            )
        pl.semaphore_wait(barrier_sem, 2)
        wq_cp.wait()
        pending = {"wo": True, "kv": [True] * N_GROUPS}

        def ensure_kv(r):
            if pending["kv"][r]:
                for cp in kv_copies[r]:
                    cp.wait()
                pending["kv"][r] = False

        def ensure_wo():
            if pending["wo"]:
                wo_cp.wait()
                pending["wo"] = False

        def compute_half(c, off):
            row0 = c * CHUNK + off
            xh = x_ref[0, pl.ds(row0, HALF), :]
            q = jnp.dot(xh, wq_sl[...],
                        preferred_element_type=jnp.float32)
            q4 = q.reshape(N_GROUPS, BLK, HQ, DH)
            rows = []
            for r in range(N_GROUPS):
                ensure_kv(r)
                qr = q4[r].transpose(1, 0, 2)
                kr = kg[r].reshape(HQ, GROUP, DH)
                vr = vg[r].reshape(HQ, GROUP, DH)
                s = lax.dot_general(
                    qr, kr, (((2,), (2,)), ((0,), (0,))),
                    preferred_element_type=jnp.float32) * SCALE
                s = s - jnp.max(s, axis=-1, keepdims=True)
                w = jnp.exp(s)
                w = w / jnp.sum(w, axis=-1, keepdims=True)
                ctx = lax.dot_general(
                    w, vr, (((2,), (1,)), ((0,), (0,))),
                    preferred_element_type=jnp.float32)
                rows.append(ctx.transpose(1, 0, 2).reshape(BLK, D_ATTN))
            ctx_half = jnp.concatenate(rows)
            ensure_wo()
            out_ref[0, pl.ds(row0, HALF), :] = jnp.dot(
                ctx_half, wo_sl[...], preferred_element_type=jnp.float32)

        def cw_chunk(k):
            return lax.rem(my + 4 * N_DEV - k, N_DEV)

        def ccw_chunk(k):
            return lax.rem(my + k, N_DEV)

        def rs_rdma(s):
            cw = pltpu.make_async_remote_copy(
                src_ref=stage_ref.at[0],
                dst_ref=comm_ref.at[s],
                send_sem=send_sems.at[s],
                recv_sem=recv_sems.at[s],
                device_id=(right,),
                device_id_type=pl.DeviceIdType.MESH,
            )
            ccw = pltpu.make_async_remote_copy(
                src_ref=stage_ref.at[1],
                dst_ref=comm_ref.at[3 + s],
                send_sem=send_sems.at[3 + s],
                recv_sem=recv_sems.at[3 + s],
                device_id=(left,),
                device_id_type=pl.DeviceIdType.MESH,
            )
            return cw, ccw

        cw_d, ccw_d = rs_rdma(0)
        compute_half(cw_chunk(0), 0)
        stage_ref[0] = out_ref[0, pl.ds(cw_chunk(0) * CHUNK, HALF), :].astype(
            jnp.bfloat16)
        cw_d.start()
        compute_half(ccw_chunk(0), HALF)
        stage_ref[1] = out_ref[
            0, pl.ds(ccw_chunk(0) * CHUNK + HALF, HALF), :].astype(jnp.bfloat16)
        ccw_d.start()

        for s in range(N_DEV - 1):
            compute_half(cw_chunk(s + 1), 0)
            compute_half(ccw_chunk(s + 1), HALF)
            cw_d.wait()
            ccw_d.wait()
            tmp_cw = (out_ref[0, pl.ds(cw_chunk(s + 1) * CHUNK, HALF), :]
                      + comm_ref[s].astype(jnp.float32))
            tmp_ccw = (out_ref[0, pl.ds(ccw_chunk(s + 1) * CHUNK + HALF, HALF), :]
                       + comm_ref[3 + s].astype(jnp.float32))
            if s < N_DEV - 2:
                stage_ref[0] = tmp_cw.astype(jnp.bfloat16)
                stage_ref[1] = tmp_ccw.astype(jnp.bfloat16)
                cw_d, ccw_d = rs_rdma(s + 1)
                cw_d.start()
                ccw_d.start()
            else:
                stage_ref[2] = tmp_cw.astype(jnp.bfloat16)
                stage_ref[3] = tmp_ccw.astype(jnp.bfloat16)
            out_ref[0, pl.ds(cw_chunk(s + 1) * CHUNK, HALF), :] = tmp_cw
            out_ref[0, pl.ds(ccw_chunk(s + 1) * CHUNK + HALF, HALF), :] = tmp_ccw

        for t in range(N_DEV - 1):
            cw = pltpu.make_async_remote_copy(
                src_ref=stage_ref.at[2] if t == 0 else comm_ref.at[t - 1],
                dst_ref=comm_ref.at[t],
                send_sem=send_sems.at[6 + t],
                recv_sem=recv_sems.at[6 + t],
                device_id=(right,),
                device_id_type=pl.DeviceIdType.MESH,
            )
            ccw = pltpu.make_async_remote_copy(
                src_ref=stage_ref.at[3] if t == 0 else comm_ref.at[3 + t - 1],
                dst_ref=comm_ref.at[3 + t],
                send_sem=send_sems.at[9 + t],
                recv_sem=recv_sems.at[9 + t],
                device_id=(left,),
                device_id_type=pl.DeviceIdType.MESH,
            )
            cw.start()
            ccw.start()
            if t > 0:
                gc = lax.rem(my + 4 * N_DEV - (t - 1), N_DEV)
                bc = lax.rem(my + t - 1, N_DEV)
                out_ref[0, pl.ds(gc * CHUNK, HALF), :] = (
                    comm_ref[t - 1].astype(jnp.float32))
                out_ref[0, pl.ds(bc * CHUNK + HALF, HALF), :] = (
                    comm_ref[3 + t - 1].astype(jnp.float32))
            cw.wait()
            ccw.wait()
        t_last = N_DEV - 2
        gc = lax.rem(my + 4 * N_DEV - t_last, N_DEV)
        bc = lax.rem(my + t_last, N_DEV)
        out_ref[0, pl.ds(gc * CHUNK, HALF), :] = (
            comm_ref[t_last].astype(jnp.float32))
        out_ref[0, pl.ds(bc * CHUNK + HALF, HALF), :] = (
            comm_ref[3 + t_last].astype(jnp.float32))

    out_shape = jax.ShapeDtypeStruct((1, SQ, D_MODEL), jnp.float32)
    return pl.pallas_call(
        body,
        out_shape=out_shape,
        in_specs=[
            pl.BlockSpec(memory_space=pltpu.VMEM),
            pl.BlockSpec(memory_space=pltpu.MemorySpace.HBM),
            pl.BlockSpec(memory_space=pltpu.MemorySpace.HBM),
            pl.BlockSpec(memory_space=pltpu.MemorySpace.HBM),
            pl.BlockSpec(memory_space=pltpu.MemorySpace.HBM),
        ],
        out_specs=pl.BlockSpec(memory_space=pltpu.VMEM),
        scratch_shapes=[
            pltpu.VMEM((D_MODEL, D_ATTN), jnp.float32),
            pltpu.VMEM((D_ATTN, D_MODEL), jnp.float32),
            pltpu.VMEM((N_GROUPS, HQ, M_BLOCKS, BLK, DH), jnp.float32),
            pltpu.VMEM((N_GROUPS, HQ, M_BLOCKS, BLK, DH), jnp.float32),
            pltpu.VMEM((6, HALF, D_MODEL), jnp.bfloat16),
            pltpu.VMEM((4, HALF, D_MODEL), jnp.bfloat16),
            pltpu.SemaphoreType.DMA((6,)),
            pltpu.SemaphoreType.DMA((12,)),
            pltpu.SemaphoreType.DMA((12,)),
        ],
        compiler_params=pltpu.CompilerParams(collective_id=0),
    )(x, Wq, K2, V2, Wo)
